# baseline (speedup 1.0000x reference)
import functools

import jax
import jax.numpy as jnp
from jax import lax
from jax.experimental import pallas as pl
from jax.experimental.pallas import tpu as pltpu
from jax.experimental.pallas import tpu_sc as plsc

B = 16384
H = 128
O = 64
T1_ROWS = 31 * 6
T2_ROWS = 26 * 6 * 3

NC, NS = 2, 16
NW = NC * NS
BPW = B // NW
CHUNK = 128
NCH = BPW // CHUNK


def _tables_body(eg, ei, em, er, el, w1, b1, t1, t2):
    f32 = jnp.float32
    mg = jnp.dot(eg[...], w1[0:8, :], preferred_element_type=f32)
    mi = jnp.dot(ei[...], w1[8:16, :], preferred_element_type=f32)
    mm = jnp.dot(em[...], w1[16:24, :], preferred_element_type=f32)
    mr = jnp.dot(er[...], w1[24:32, :], preferred_element_type=f32)
    ml = jnp.dot(el[...], w1[32:40, :], preferred_element_type=f32)

    r1 = lax.broadcasted_iota(jnp.int32, (T1_ROWS, 1), 0)
    orr = (r1 // 6 == lax.broadcasted_iota(jnp.int32, (T1_ROWS, 31), 1)).astype(f32)
    om = (r1 % 6 == lax.broadcasted_iota(jnp.int32, (T1_ROWS, 6), 1)).astype(f32)
    t1[...] = (jnp.dot(orr, mr, preferred_element_type=f32)
               + jnp.dot(om, mm, preferred_element_type=f32)
               + b1[...])

    r2 = lax.broadcasted_iota(jnp.int32, (T2_ROWS, 1), 0)
    ol = (r2 // 18 == lax.broadcasted_iota(jnp.int32, (T2_ROWS, 26), 1)).astype(f32)
    oi = ((r2 // 3) % 6 == lax.broadcasted_iota(jnp.int32, (T2_ROWS, 6), 1)).astype(f32)
    og = (r2 % 3 == lax.broadcasted_iota(jnp.int32, (T2_ROWS, 3), 1)).astype(f32)
    t2[...] = (jnp.dot(ol, ml, preferred_element_type=f32)
               + jnp.dot(oi, mi, preferred_element_type=f32)
               + jnp.dot(og, mg, preferred_element_type=f32))


_build_tables = pl.pallas_call(
    _tables_body,
    out_shape=(jax.ShapeDtypeStruct((T1_ROWS, H), jnp.float32),
               jax.ShapeDtypeStruct((T2_ROWS, H), jnp.float32)),
)


def _gather_body(g, i, m, r, l, t1, t2, out,
                 ta, tb, gv, iv, mv, rv, lv, idx1, idx2, wbs,
                 stab, sidx, sw):
    wid = lax.axis_index("s") * NC + lax.axis_index("c")
    base = wid * BPW

    cpa = pltpu.async_copy(t1, ta, stab)
    cpb = pltpu.async_copy(t2, tb, stab)
    cps = [pltpu.async_copy(src.at[pl.ds(base, BPW)], dst, sidx)
           for src, dst in ((g, gv), (i, iv), (m, mv), (r, rv), (l, lv))]
    for cp in cps:
        cp.wait()

    @plsc.parallel_loop(0, BPW // 16, unroll=4)
    def icomp(k):
        s = pl.ds(k * 16, 16)
        idx1[s] = rv[s] * 6 + mv[s]
        idx2[s] = lv[s] * 18 + iv[s] * 3 + gv[s]

    cpa.wait()
    cpb.wait()

    cols = [lax.iota(jnp.int32, 16) + (j * 16) for j in range(H // 16)]

    wr = [None, None]
    for c in range(NCH):
        s = c % 2
        if wr[s] is not None:
            wr[s].wait()

        @plsc.parallel_loop(0, CHUNK, unroll=4)
        def row(rr):
            pos = jnp.full((16,), c * CHUNK + rr, jnp.int32)
            ra = plsc.load_gather(idx1, [pos])
            rb = plsc.load_gather(idx2, [pos])
            for j in range(H // 16):
                js = pl.ds(j * 16, 16)
                va = plsc.load_gather(ta, [ra, cols[j]])
                vb = plsc.load_gather(tb, [rb, cols[j]])
                wbs[s, rr, js] = va + vb

        wr[s] = pltpu.async_copy(wbs.at[s], out.at[pl.ds(base + c * CHUNK, CHUNK)],
                                 sw.at[s])
    wr[0].wait()
    wr[1].wait()


@functools.cache
def _make_gather_sum():
  return functools.partial(
    pl.kernel,
    out_type=jax.ShapeDtypeStruct((B, H), jnp.float32),
    mesh=plsc.VectorSubcoreMesh(core_axis_name="c", subcore_axis_name="s",
                                num_cores=NC, num_subcores=NS),
    compiler_params=pltpu.CompilerParams(needs_layout_passes=False),
    scratch_types=[
        pltpu.VMEM((T1_ROWS, H), jnp.float32),
        pltpu.VMEM((T2_ROWS, H), jnp.float32),
        pltpu.VMEM((BPW,), jnp.int32),
        pltpu.VMEM((BPW,), jnp.int32),
        pltpu.VMEM((BPW,), jnp.int32),
        pltpu.VMEM((BPW,), jnp.int32),
        pltpu.VMEM((BPW,), jnp.int32),
        pltpu.VMEM((BPW,), jnp.int32),
        pltpu.VMEM((BPW,), jnp.int32),
        pltpu.VMEM((2, CHUNK, H), jnp.float32),
        pltpu.SemaphoreType.DMA,
        pltpu.SemaphoreType.DMA,
        pltpu.SemaphoreType.DMA((2,)),
    ],
  )(_gather_body)


def _mlp_body(hp, sc, w1r, lg, lb, w2, b2, out):
    x = hp[...] + sc[...] * w1r[...]
    mu = jnp.mean(x, axis=-1, keepdims=True)
    d = x - mu
    var = jnp.mean(d * d, axis=-1, keepdims=True)
    y = d * lax.rsqrt(var + 1e-5) * lg[...] + lb[...]
    y = jnp.maximum(y, 0.0)
    out[...] = jnp.dot(y, w2[...], preferred_element_type=jnp.float32) + b2[...]


_MLP_BS = 8192


def _mlp(hp, scal, w1row, ln_g, ln_b, W2, b2):
    grid = (B // _MLP_BS,)
    return pl.pallas_call(
        _mlp_body,
        grid=grid,
        in_specs=[
            pl.BlockSpec((_MLP_BS, H), lambda n: (n, 0)),
            pl.BlockSpec((_MLP_BS, 1), lambda n: (n, 0)),
            pl.BlockSpec((1, H), lambda n: (0, 0)),
            pl.BlockSpec((1, H), lambda n: (0, 0)),
            pl.BlockSpec((1, H), lambda n: (0, 0)),
            pl.BlockSpec((H, O), lambda n: (0, 0)),
            pl.BlockSpec((1, O), lambda n: (0, 0)),
        ],
        out_specs=pl.BlockSpec((_MLP_BS, O), lambda n: (n, 0)),
        out_shape=jax.ShapeDtypeStruct((B, O), jnp.float32),
    )(hp, scal, w1row, ln_g, ln_b, W2, b2)


def kernel(gender, insurance, marital_status, race, language, scalar_inputs,
           emb_gender, emb_insurance, emb_marital_status, emb_race, emb_language,
           W1, b1, ln_g, ln_b, W2, b2):
    g = gender.astype(jnp.int32)
    i = insurance.astype(jnp.int32)
    m = marital_status.astype(jnp.int32)
    r = race.astype(jnp.int32)
    l = language.astype(jnp.int32)

    t1, t2 = _build_tables(emb_gender, emb_insurance, emb_marital_status,
                           emb_race, emb_language, W1, b1.reshape(1, H))
    hp = _make_gather_sum()(g, i, m, r, l, t1, t2)
    return _mlp(hp, scalar_inputs, W1[40].reshape(1, H),
                ln_g.reshape(1, H), ln_b.reshape(1, H), W2, b2.reshape(1, O))


# speedup vs baseline: 1.0014x; 1.0014x over previous
import functools

import jax
import jax.numpy as jnp
from jax import lax
from jax.experimental import pallas as pl
from jax.experimental.pallas import tpu as pltpu
from jax.experimental.pallas import tpu_sc as plsc

B = 16384
H = 128
O = 64
T1_ROWS = 31 * 6
T2_ROWS = 26 * 6 * 3

NC, NS = 2, 16
NW = NC * NS
BPW = B // NW
CHUNK = 128
NCH = BPW // CHUNK


def _tables_body(eg, ei, em, er, el, w1, b1, t1, t2):
    f32 = jnp.float32
    mg = jnp.dot(eg[...], w1[0:8, :], preferred_element_type=f32)
    mi = jnp.dot(ei[...], w1[8:16, :], preferred_element_type=f32)
    mm = jnp.dot(em[...], w1[16:24, :], preferred_element_type=f32)
    mr = jnp.dot(er[...], w1[24:32, :], preferred_element_type=f32)
    ml = jnp.dot(el[...], w1[32:40, :], preferred_element_type=f32)

    r1 = lax.broadcasted_iota(jnp.int32, (T1_ROWS, 1), 0)
    orr = (r1 // 6 == lax.broadcasted_iota(jnp.int32, (T1_ROWS, 31), 1)).astype(f32)
    om = (r1 % 6 == lax.broadcasted_iota(jnp.int32, (T1_ROWS, 6), 1)).astype(f32)
    t1[...] = (jnp.dot(orr, mr, preferred_element_type=f32)
               + jnp.dot(om, mm, preferred_element_type=f32)
               + b1[...])

    r2 = lax.broadcasted_iota(jnp.int32, (T2_ROWS, 1), 0)
    ol = (r2 // 18 == lax.broadcasted_iota(jnp.int32, (T2_ROWS, 26), 1)).astype(f32)
    oi = ((r2 // 3) % 6 == lax.broadcasted_iota(jnp.int32, (T2_ROWS, 6), 1)).astype(f32)
    og = (r2 % 3 == lax.broadcasted_iota(jnp.int32, (T2_ROWS, 3), 1)).astype(f32)
    t2[...] = (jnp.dot(ol, ml, preferred_element_type=f32)
               + jnp.dot(oi, mi, preferred_element_type=f32)
               + jnp.dot(og, mg, preferred_element_type=f32))


_build_tables = pl.pallas_call(
    _tables_body,
    out_shape=(jax.ShapeDtypeStruct((T1_ROWS, H), jnp.float32),
               jax.ShapeDtypeStruct((T2_ROWS, H), jnp.float32)),
)


def _gather_body(g, i, m, r, l, t1, t2, out,
                 ta, tb, gv, iv, mv, rv, lv, idx1, idx2, wbs,
                 stab, sidx, sw):
    wid = lax.axis_index("s") * NC + lax.axis_index("c")
    base = wid * BPW

    cpa = pltpu.async_copy(t1, ta, stab)
    cpb = pltpu.async_copy(t2, tb, stab)
    cps = [pltpu.async_copy(src.at[pl.ds(base, BPW)], dst, sidx)
           for src, dst in ((g, gv), (i, iv), (m, mv), (r, rv), (l, lv))]
    for cp in cps:
        cp.wait()

    @plsc.parallel_loop(0, BPW // 16, unroll=4)
    def icomp(k):
        s = pl.ds(k * 16, 16)
        idx1[s] = rv[s] * 6 + mv[s]
        idx2[s] = lv[s] * 18 + iv[s] * 3 + gv[s]

    cpa.wait()
    cpb.wait()

    cols = [lax.iota(jnp.int32, 16) + (j * 16) for j in range(H // 16)]

    wr = [None, None]
    for c in range(NCH):
        s = c % 2
        if wr[s] is not None:
            wr[s].wait()

        @plsc.parallel_loop(0, CHUNK, unroll=4)
        def row(rr):
            pos = jnp.full((16,), c * CHUNK + rr, jnp.int32)
            ra = plsc.load_gather(idx1, [pos])
            rb = plsc.load_gather(idx2, [pos])
            for j in range(H // 16):
                js = pl.ds(j * 16, 16)
                va = plsc.load_gather(ta, [ra, cols[j]])
                vb = plsc.load_gather(tb, [rb, cols[j]])
                wbs[s, rr, js] = va + vb

        wr[s] = pltpu.async_copy(wbs.at[s], out.at[pl.ds(base + c * CHUNK, CHUNK)],
                                 sw.at[s])
    wr[0].wait()
    wr[1].wait()


@functools.cache
def _make_gather_sum():
  return functools.partial(
    pl.kernel,
    out_type=jax.ShapeDtypeStruct((B, H), jnp.float32),
    mesh=plsc.VectorSubcoreMesh(core_axis_name="c", subcore_axis_name="s",
                                num_cores=NC, num_subcores=NS),
    compiler_params=pltpu.CompilerParams(needs_layout_passes=False),
    scratch_types=[
        pltpu.VMEM((T1_ROWS, H), jnp.float32),
        pltpu.VMEM((T2_ROWS, H), jnp.float32),
        pltpu.VMEM((BPW,), jnp.int32),
        pltpu.VMEM((BPW,), jnp.int32),
        pltpu.VMEM((BPW,), jnp.int32),
        pltpu.VMEM((BPW,), jnp.int32),
        pltpu.VMEM((BPW,), jnp.int32),
        pltpu.VMEM((BPW,), jnp.int32),
        pltpu.VMEM((BPW,), jnp.int32),
        pltpu.VMEM((2, CHUNK, H), jnp.float32),
        pltpu.SemaphoreType.DMA,
        pltpu.SemaphoreType.DMA,
        pltpu.SemaphoreType.DMA((2,)),
    ],
  )(_gather_body)


def _mlp_body(hp, sc, w1r, lg, lb, w2, b2, out):
    x = hp[...] + sc[...] * w1r[...]
    mu = jnp.mean(x, axis=-1, keepdims=True)
    d = x - mu
    var = jnp.mean(d * d, axis=-1, keepdims=True)
    y = d * lax.rsqrt(var + 1e-5) * lg[...] + lb[...]
    y = jnp.maximum(y, 0.0)
    out[...] = jnp.dot(y, w2[...], preferred_element_type=jnp.float32) + b2[...]


_MLP_BS = 4096


def _mlp(hp, scal, w1row, ln_g, ln_b, W2, b2):
    grid = (B // _MLP_BS,)
    return pl.pallas_call(
        _mlp_body,
        grid=grid,
        in_specs=[
            pl.BlockSpec((_MLP_BS, H), lambda n: (n, 0)),
            pl.BlockSpec((_MLP_BS, 1), lambda n: (n, 0)),
            pl.BlockSpec((1, H), lambda n: (0, 0)),
            pl.BlockSpec((1, H), lambda n: (0, 0)),
            pl.BlockSpec((1, H), lambda n: (0, 0)),
            pl.BlockSpec((H, O), lambda n: (0, 0)),
            pl.BlockSpec((1, O), lambda n: (0, 0)),
        ],
        out_specs=pl.BlockSpec((_MLP_BS, O), lambda n: (n, 0)),
        out_shape=jax.ShapeDtypeStruct((B, O), jnp.float32),
    )(hp, scal, w1row, ln_g, ln_b, W2, b2)


def kernel(gender, insurance, marital_status, race, language, scalar_inputs,
           emb_gender, emb_insurance, emb_marital_status, emb_race, emb_language,
           W1, b1, ln_g, ln_b, W2, b2):
    g = gender.astype(jnp.int32)
    i = insurance.astype(jnp.int32)
    m = marital_status.astype(jnp.int32)
    r = race.astype(jnp.int32)
    l = language.astype(jnp.int32)

    t1, t2 = _build_tables(emb_gender, emb_insurance, emb_marital_status,
                           emb_race, emb_language, W1, b1.reshape(1, H))
    hp = _make_gather_sum()(g, i, m, r, l, t1, t2)
    return _mlp(hp, scalar_inputs, W1[40].reshape(1, H),
                ln_g.reshape(1, H), ln_b.reshape(1, H), W2, b2.reshape(1, O))


# speedup vs baseline: 1.1442x; 1.1426x over previous
import functools

import jax
import jax.numpy as jnp
from jax import lax
from jax.experimental import pallas as pl
from jax.experimental.pallas import tpu as pltpu
from jax.experimental.pallas import tpu_sc as plsc

B = 16384
H = 128
O = 64
T1_ROWS = 31 * 6
T2_ROWS = 26 * 6 * 3

NC, NS = 2, 16
NW = NC * NS
BPW = B // NW
CHUNK = 128
NCH = BPW // CHUNK


def _tables_body(eg, ei, em, er, el, w1, b1, t1, t2):
    f32 = jnp.float32
    mg = jnp.dot(eg[...], w1[0:8, :], preferred_element_type=f32)
    mi = jnp.dot(ei[...], w1[8:16, :], preferred_element_type=f32)
    mm = jnp.dot(em[...], w1[16:24, :], preferred_element_type=f32)
    mr = jnp.dot(er[...], w1[24:32, :], preferred_element_type=f32)
    ml = jnp.dot(el[...], w1[32:40, :], preferred_element_type=f32)

    r1 = lax.broadcasted_iota(jnp.int32, (T1_ROWS, 1), 0)
    orr = (r1 // 6 == lax.broadcasted_iota(jnp.int32, (T1_ROWS, 31), 1)).astype(f32)
    om = (r1 % 6 == lax.broadcasted_iota(jnp.int32, (T1_ROWS, 6), 1)).astype(f32)
    t1[...] = (jnp.dot(orr, mr, preferred_element_type=f32)
               + jnp.dot(om, mm, preferred_element_type=f32)
               + b1[...])

    r2 = lax.broadcasted_iota(jnp.int32, (T2_ROWS, 1), 0)
    ol = (r2 // 18 == lax.broadcasted_iota(jnp.int32, (T2_ROWS, 26), 1)).astype(f32)
    oi = ((r2 // 3) % 6 == lax.broadcasted_iota(jnp.int32, (T2_ROWS, 6), 1)).astype(f32)
    og = (r2 % 3 == lax.broadcasted_iota(jnp.int32, (T2_ROWS, 3), 1)).astype(f32)
    t2[...] = (jnp.dot(ol, ml, preferred_element_type=f32)
               + jnp.dot(oi, mi, preferred_element_type=f32)
               + jnp.dot(og, mg, preferred_element_type=f32))


_build_tables = pl.pallas_call(
    _tables_body,
    out_shape=(jax.ShapeDtypeStruct((T1_ROWS, H), jnp.float32),
               jax.ShapeDtypeStruct((T2_ROWS, H), jnp.float32)),
)


def _gather_body(g, i, m, r, l, t1, t2, out,
                 ta, tb, t1s, t2s, gv, iv, mv, rv, lv, idx1, idx2, wbs,
                 stab, sidx, sw):
    sid = lax.axis_index("s")
    wid = sid * NC + lax.axis_index("c")
    base = wid * BPW

    @pl.when(sid == 0)
    def _load_shared():
        pltpu.sync_copy(t1, t1s)
        pltpu.sync_copy(t2, t2s)
    plsc.subcore_barrier()
    cpa = pltpu.async_copy(t1s, ta, stab)
    cpb = pltpu.async_copy(t2s, tb, stab)
    cps = [pltpu.async_copy(src.at[pl.ds(base, BPW)], dst, sidx)
           for src, dst in ((g, gv), (i, iv), (m, mv), (r, rv), (l, lv))]
    for cp in cps:
        cp.wait()

    @plsc.parallel_loop(0, BPW // 16, unroll=4)
    def icomp(k):
        s = pl.ds(k * 16, 16)
        idx1[s] = rv[s] * 6 + mv[s]
        idx2[s] = lv[s] * 18 + iv[s] * 3 + gv[s]

    cpa.wait()
    cpb.wait()

    cols = [lax.iota(jnp.int32, 16) + (j * 16) for j in range(H // 16)]

    wr = [None, None]
    for c in range(NCH):
        s = c % 2
        if wr[s] is not None:
            wr[s].wait()

        @plsc.parallel_loop(0, CHUNK, unroll=4)
        def row(rr):
            pos = jnp.full((16,), c * CHUNK + rr, jnp.int32)
            ra = plsc.load_gather(idx1, [pos])
            rb = plsc.load_gather(idx2, [pos])
            for j in range(H // 16):
                js = pl.ds(j * 16, 16)
                va = plsc.load_gather(ta, [ra, cols[j]])
                vb = plsc.load_gather(tb, [rb, cols[j]])
                wbs[s, rr, js] = va + vb

        wr[s] = pltpu.async_copy(wbs.at[s], out.at[pl.ds(base + c * CHUNK, CHUNK)],
                                 sw.at[s])
    wr[0].wait()
    wr[1].wait()


@functools.cache
def _make_gather_sum():
  return functools.partial(
    pl.kernel,
    out_type=jax.ShapeDtypeStruct((B, H), jnp.float32),
    mesh=plsc.VectorSubcoreMesh(core_axis_name="c", subcore_axis_name="s",
                                num_cores=NC, num_subcores=NS),
    compiler_params=pltpu.CompilerParams(needs_layout_passes=False),
    scratch_types=[
        pltpu.VMEM((T1_ROWS, H), jnp.float32),
        pltpu.VMEM((T2_ROWS, H), jnp.float32),
        pltpu.VMEM_SHARED((T1_ROWS, H), jnp.float32),
        pltpu.VMEM_SHARED((T2_ROWS, H), jnp.float32),
        pltpu.VMEM((BPW,), jnp.int32),
        pltpu.VMEM((BPW,), jnp.int32),
        pltpu.VMEM((BPW,), jnp.int32),
        pltpu.VMEM((BPW,), jnp.int32),
        pltpu.VMEM((BPW,), jnp.int32),
        pltpu.VMEM((BPW,), jnp.int32),
        pltpu.VMEM((BPW,), jnp.int32),
        pltpu.VMEM((2, CHUNK, H), jnp.float32),
        pltpu.SemaphoreType.DMA,
        pltpu.SemaphoreType.DMA,
        pltpu.SemaphoreType.DMA((2,)),
    ],
  )(_gather_body)


def _mlp_body(hp, sc, w1r, lg, lb, w2, b2, out):
    x = hp[...] + sc[...] * w1r[...]
    mu = jnp.mean(x, axis=-1, keepdims=True)
    d = x - mu
    var = jnp.mean(d * d, axis=-1, keepdims=True)
    y = d * lax.rsqrt(var + 1e-5) * lg[...] + lb[...]
    y = jnp.maximum(y, 0.0)
    out[...] = jnp.dot(y, w2[...], preferred_element_type=jnp.float32) + b2[...]


_MLP_BS = 4096


def _mlp(hp, scal, w1row, ln_g, ln_b, W2, b2):
    grid = (B // _MLP_BS,)
    return pl.pallas_call(
        _mlp_body,
        grid=grid,
        in_specs=[
            pl.BlockSpec((_MLP_BS, H), lambda n: (n, 0)),
            pl.BlockSpec((_MLP_BS, 1), lambda n: (n, 0)),
            pl.BlockSpec((1, H), lambda n: (0, 0)),
            pl.BlockSpec((1, H), lambda n: (0, 0)),
            pl.BlockSpec((1, H), lambda n: (0, 0)),
            pl.BlockSpec((H, O), lambda n: (0, 0)),
            pl.BlockSpec((1, O), lambda n: (0, 0)),
        ],
        out_specs=pl.BlockSpec((_MLP_BS, O), lambda n: (n, 0)),
        out_shape=jax.ShapeDtypeStruct((B, O), jnp.float32),
    )(hp, scal, w1row, ln_g, ln_b, W2, b2)


def kernel(gender, insurance, marital_status, race, language, scalar_inputs,
           emb_gender, emb_insurance, emb_marital_status, emb_race, emb_language,
           W1, b1, ln_g, ln_b, W2, b2):
    g = gender.astype(jnp.int32)
    i = insurance.astype(jnp.int32)
    m = marital_status.astype(jnp.int32)
    r = race.astype(jnp.int32)
    l = language.astype(jnp.int32)

    t1, t2 = _build_tables(emb_gender, emb_insurance, emb_marital_status,
                           emb_race, emb_language, W1, b1.reshape(1, H))
    hp = _make_gather_sum()(g, i, m, r, l, t1, t2)
    return _mlp(hp, scalar_inputs, W1[40].reshape(1, H),
                ln_g.reshape(1, H), ln_b.reshape(1, H), W2, b2.reshape(1, O))
